# baseline (speedup 1.0000x reference)
import functools

import jax
import jax.numpy as jnp
from jax import lax
from jax.experimental import pallas as pl
from jax.experimental.pallas import tpu as pltpu
from jax.experimental.pallas import tpu_sc as plsc

_NC = 2
_NS = 16
_NW = _NC * _NS

_IDX_W = 88


@functools.partial(jax.jit, static_argnames=("n_ctx", "suf_len", "d"))
def _sc_gather(prefix2d, ctx2d, suffix2d, idx_flat, *, n_ctx, suf_len, d):
    b = idx_flat.shape[0] // _IDX_W
    nb = b // _NW
    seq = 1 + n_ctx + suf_len
    mesh = plsc.VectorSubcoreMesh(
        core_axis_name="c", subcore_axis_name="s",
        num_cores=_NC, num_subcores=_NS,
    )

    @functools.partial(
        pl.kernel,
        out_type=jax.ShapeDtypeStruct((b, seq, d), jnp.float32),
        mesh=mesh,
        compiler_params=pltpu.CompilerParams(use_tc_tiling_on_sc=False),
        scratch_types=[
            pltpu.VMEM((nb * _IDX_W,), jnp.int32),
            pltpu.VMEM((seq, d), jnp.float32),
            pltpu.VMEM((seq, d), jnp.float32),
            pltpu.SemaphoreType.DMA,
            pltpu.SemaphoreType.DMA,
            pltpu.SemaphoreType.DMA,
            pltpu.SemaphoreType.DMA,
        ],
    )
    def k(pre_hbm, ctx_hbm, suf_hbm, idx_hbm, out_hbm,
          idx_v, buf0, buf1, g0, g1, s0, s1):
        wid = lax.axis_index("s") * _NC + lax.axis_index("c")
        base = wid * nb
        pltpu.sync_copy(
            idx_hbm.at[pl.ds(pl.multiple_of(base * _IDX_W, 8), nb * _IDX_W)],
            idx_v)

        bufs = (buf0, buf1)
        gsems = (g0, g1)
        ssems = (s0, s1)

        def fire(i, p):
            off = pl.multiple_of(i * _IDX_W, 8)
            buf = bufs[p]
            sem = gsems[p]
            pltpu.async_copy(
                suf_hbm.at[idx_v.at[pl.ds(off, suf_len)]],
                buf.at[pl.ds(1 + n_ctx, suf_len)], sem)
            pltpu.async_copy(
                ctx_hbm.at[idx_v.at[pl.ds(off + 64, n_ctx)]],
                buf.at[pl.ds(1, n_ctx)], sem)
            pltpu.async_copy(
                pre_hbm.at[idx_v.at[pl.ds(off + 80, 1)]],
                buf.at[pl.ds(0, 1)], sem)

        def wait_gathers(i, p):
            off = pl.multiple_of(i * _IDX_W, 8)
            buf = bufs[p]
            sem = gsems[p]
            pltpu.make_async_copy(
                suf_hbm.at[idx_v.at[pl.ds(off, suf_len)]],
                buf.at[pl.ds(1 + n_ctx, suf_len)], sem).wait()
            pltpu.make_async_copy(
                ctx_hbm.at[idx_v.at[pl.ds(off + 64, n_ctx)]],
                buf.at[pl.ds(1, n_ctx)], sem).wait()
            pltpu.make_async_copy(
                pre_hbm.at[idx_v.at[pl.ds(off + 80, 1)]],
                buf.at[pl.ds(0, 1)], sem).wait()

        def store(i, p):
            pltpu.async_copy(bufs[p], out_hbm.at[base + i], ssems[p])

        def wait_store(i, p):
            pltpu.make_async_copy(bufs[p], out_hbm.at[base + i], ssems[p]).wait()

        fire(0, 0)

        def body(j, _):
            i = j * 2
            @pl.when(j > 0)
            def _():
                wait_store(i - 1, 1)
            fire(i + 1, 1)
            wait_gathers(i, 0)
            store(i, 0)
            @pl.when(j < nb // 2 - 1)
            def _():
                wait_store(i, 0)
                fire(i + 2, 0)
            wait_gathers(i + 1, 1)
            store(i + 1, 1)
            return 0

        lax.fori_loop(0, nb // 2, body, 0)
        wait_store(nb - 2, 0)
        wait_store(nb - 1, 1)

    return k(prefix2d, ctx2d, suffix2d, idx_flat)


def kernel(labels, ctx, token_prefix, token_suffix):
    n_cls, n_ctx, d = ctx.shape
    suf_len = token_suffix.shape[1]
    b = labels.shape[0]
    lab = labels.astype(jnp.int32)

    suf_idx = lab[:, None] * suf_len + jnp.arange(suf_len, dtype=jnp.int32)
    ctx_idx = lab[:, None] * n_ctx + jnp.arange(n_ctx, dtype=jnp.int32)
    pad4 = jnp.zeros((b, 4), jnp.int32)
    pad7 = jnp.zeros((b, 7), jnp.int32)
    idx = jnp.concatenate(
        [suf_idx, pad4, ctx_idx, lab[:, None], pad7], axis=1)

    return _sc_gather(
        token_prefix.reshape(n_cls, d),
        ctx.reshape(n_cls * n_ctx, d),
        token_suffix.reshape(n_cls * suf_len, d),
        idx.reshape(-1),
        n_ctx=n_ctx, suf_len=suf_len, d=d)


# speedup vs baseline: 1.0978x; 1.0978x over previous
import functools

import jax
import jax.numpy as jnp
from jax import lax
from jax.experimental import pallas as pl
from jax.experimental.pallas import tpu as pltpu
from jax.experimental.pallas import tpu_sc as plsc

_NC = 2
_NS = 16
_NW = _NC * _NS

_IDX_W = 8


@functools.partial(jax.jit, static_argnames=("n_ctx", "suf_len", "d"))
def _sc_gather(prefix2d, ctx2d, suffix2d, idx_flat, *, n_ctx, suf_len, d):
    b = idx_flat.shape[0] // _IDX_W
    nb = b // _NW
    seq = 1 + n_ctx + suf_len
    row = seq * d
    mesh = plsc.VectorSubcoreMesh(
        core_axis_name="c", subcore_axis_name="s",
        num_cores=_NC, num_subcores=_NS,
    )

    @functools.partial(
        pl.kernel,
        out_type=jax.ShapeDtypeStruct((b, row), jnp.float32),
        mesh=mesh,
        compiler_params=pltpu.CompilerParams(use_tc_tiling_on_sc=False),
        scratch_types=[
            pltpu.VMEM((nb * _IDX_W,), jnp.int32),
            pltpu.VMEM((1, row), jnp.float32),
            pltpu.VMEM((1, row), jnp.float32),
            pltpu.SemaphoreType.DMA,
            pltpu.SemaphoreType.DMA,
            pltpu.SemaphoreType.DMA,
            pltpu.SemaphoreType.DMA,
        ],
    )
    def k(pre_hbm, ctx_hbm, suf_hbm, idx_hbm, out_hbm,
          idx_v, buf0, buf1, g0, g1, s0, s1):
        wid = lax.axis_index("s") * _NC + lax.axis_index("c")
        base = wid * nb
        pltpu.sync_copy(
            idx_hbm.at[pl.ds(pl.multiple_of(base * _IDX_W, 8), nb * _IDX_W)],
            idx_v)

        bufs = (buf0, buf1)
        gsems = (g0, g1)
        ssems = (s0, s1)

        def copies(i, p):
            off = pl.multiple_of(i * _IDX_W, 8)
            lab = idx_v.at[pl.ds(off, 1)]
            buf = bufs[p]
            sem = gsems[p]
            return (
                pltpu.make_async_copy(
                    suf_hbm.at[lab], buf.at[:, pl.ds((1 + n_ctx) * d, suf_len * d)], sem),
                pltpu.make_async_copy(
                    ctx_hbm.at[lab], buf.at[:, pl.ds(d, n_ctx * d)], sem),
                pltpu.make_async_copy(
                    pre_hbm.at[lab], buf.at[:, pl.ds(0, d)], sem),
            )

        def fire(i, p):
            for c in copies(i, p):
                c.start()

        def wait_gathers(i, p):
            for c in copies(i, p):
                c.wait()

        def store(i, p):
            pltpu.async_copy(bufs[p], out_hbm.at[pl.ds(base + i, 1)], ssems[p])

        def wait_store(i, p):
            pltpu.make_async_copy(
                bufs[p], out_hbm.at[pl.ds(base + i, 1)], ssems[p]).wait()

        fire(0, 0)

        def body(j, _):
            i = j * 2
            @pl.when(j > 0)
            def _():
                wait_store(i - 1, 1)
            fire(i + 1, 1)
            wait_gathers(i, 0)
            store(i, 0)
            @pl.when(j < nb // 2 - 1)
            def _():
                wait_store(i, 0)
                fire(i + 2, 0)
            wait_gathers(i + 1, 1)
            store(i + 1, 1)
            return 0

        lax.fori_loop(0, nb // 2, body, 0)
        wait_store(nb - 2, 0)
        wait_store(nb - 1, 1)

    return k(prefix2d, ctx2d, suffix2d, idx_flat)


def kernel(labels, ctx, token_prefix, token_suffix):
    n_cls, n_ctx, d = ctx.shape
    suf_len = token_suffix.shape[1]
    b = labels.shape[0]
    seq = 1 + n_ctx + suf_len
    lab = labels.astype(jnp.int32)

    idx = jnp.zeros((b, _IDX_W), jnp.int32).at[:, 0].set(lab)

    out = _sc_gather(
        token_prefix.reshape(n_cls, d),
        ctx.reshape(n_cls, n_ctx * d),
        token_suffix.reshape(n_cls, suf_len * d),
        idx.reshape(-1),
        n_ctx=n_ctx, suf_len=suf_len, d=d)
    return out.reshape(b, seq, d)
